# baseline (speedup 1.0000x reference)
import jax
import jax.numpy as jnp
from jax.experimental import pallas as pl
from jax.experimental.pallas import tpu as pltpu

ROWS, COLS = 128, 100000
BLOCK_ROWS = 8


def _rotl(x, d):
    return (x << jnp.uint32(d)) | (x >> jnp.uint32(32 - d))


def _threefry_bits(x0, x1):
    ks0 = jnp.uint32(0)
    ks1 = jnp.uint32(1)
    ks2 = jnp.uint32(0x1BD11BDB)
    rotations = ((13, 15, 26, 6), (17, 29, 16, 24))
    ks = (ks0, ks1, ks2)
    x0 = x0 + ks0
    x1 = x1 + ks1
    for i in range(5):
        for r in rotations[i % 2]:
            x0 = x0 + x1
            x1 = _rotl(x1, r)
            x1 = x0 ^ x1
        x0 = x0 + ks[(i + 1) % 3]
        x1 = x1 + ks[(i + 2) % 3] + jnp.uint32(i + 1)
    return x0 ^ x1


def _gumbel_softmax_kernel(inv_t_ref, logits_ref, out_ref):
    b = pl.program_id(0)
    shape = logits_ref.shape

    row = jax.lax.broadcasted_iota(jnp.uint32, shape, 0)
    col = jax.lax.broadcasted_iota(jnp.uint32, shape, 1)
    base = jnp.uint32(b * BLOCK_ROWS) * jnp.uint32(COLS)
    idx = base + row * jnp.uint32(COLS) + col

    bits = _threefry_bits(jnp.zeros(shape, jnp.uint32), idx)

    fbits = (bits >> jnp.uint32(9)) | jnp.uint32(0x3F800000)
    u = jax.lax.bitcast_convert_type(fbits, jnp.float32) - 1.0

    eps = jnp.float32(1e-20)
    g = -jnp.log(-jnp.log(u + eps) + eps)

    z = (logits_ref[...] + g) * inv_t_ref[0]
    m = jnp.max(z, axis=-1, keepdims=True)
    e = jnp.exp(z - m)
    s = jnp.sum(e, axis=-1, keepdims=True)
    out_ref[...] = e / s


@jax.jit
def kernel(logits, temperature):
    inv_t = (1.0 / jnp.asarray(temperature, jnp.float32)).reshape(1)
    grid = (ROWS // BLOCK_ROWS,)
    return pl.pallas_call(
        _gumbel_softmax_kernel,
        grid=grid,
        in_specs=[
            pl.BlockSpec(memory_space=pltpu.SMEM),
            pl.BlockSpec((BLOCK_ROWS, COLS), lambda b: (b, 0)),
        ],
        out_specs=pl.BlockSpec((BLOCK_ROWS, COLS), lambda b: (b, 0)),
        out_shape=jax.ShapeDtypeStruct((ROWS, COLS), jnp.float32),
    )(inv_t, logits)


# speedup vs baseline: 1.0229x; 1.0229x over previous
import jax
import jax.numpy as jnp
from jax.experimental import pallas as pl
from jax.experimental.pallas import tpu as pltpu

ROWS, COLS = 128, 100000
BLOCK_ROWS = 8
CHUNK = 2048
NFULL = COLS // CHUNK
TAIL_START = NFULL * CHUNK
TAIL = COLS - TAIL_START


def _rotl(x, d):
    return (x << jnp.uint32(d)) | (x >> jnp.uint32(32 - d))


def _bits_from_idx(idx):
    ks0 = jnp.uint32(0)
    ks1 = jnp.uint32(1)
    ks2 = jnp.uint32(0x1BD11BDB)
    ks = (ks0, ks1, ks2)
    rotations = ((13, 15, 26, 6), (17, 29, 16, 24))
    x1 = idx + ks1
    x0 = x1
    x1 = _rotl(x1, 13)
    x1 = x0 ^ x1
    for r in (15, 26, 6):
        x0 = x0 + x1
        x1 = _rotl(x1, r)
        x1 = x0 ^ x1
    x0 = x0 + ks[1]
    x1 = x1 + ks[2] + jnp.uint32(1)
    for i in range(1, 5):
        for r in rotations[i % 2]:
            x0 = x0 + x1
            x1 = _rotl(x1, r)
            x1 = x0 ^ x1
        x0 = x0 + ks[(i + 1) % 3]
        x1 = x1 + ks[(i + 2) % 3] + jnp.uint32(i + 1)
    return x0 ^ x1


def _gumbel(idx):
    bits = _bits_from_idx(idx)
    fbits = (bits >> jnp.uint32(9)) | jnp.uint32(0x3F800000)
    u = jax.lax.bitcast_convert_type(fbits, jnp.float32) - 1.0
    eps = jnp.float32(1e-20)
    return -jnp.log(-jnp.log(u + eps) + eps)


def _gumbel_softmax_kernel(inv_t_ref, logits_ref, out_ref):
    b = pl.program_id(0)
    inv_t = inv_t_ref[0]
    base = jnp.uint32(b) * jnp.uint32(BLOCK_ROWS * COLS)

    def idx_chunk(start, width):
        shape = (BLOCK_ROWS, width)
        row = jax.lax.broadcasted_iota(jnp.uint32, shape, 0)
        col = jax.lax.broadcasted_iota(jnp.uint32, shape, 1)
        start_u = jnp.asarray(start, jnp.int32).astype(jnp.uint32)
        return base + row * jnp.uint32(COLS) + (col + start_u)

    def z_chunk(start, width):
        sl = (slice(None), pl.ds(start, width))
        g = _gumbel(idx_chunk(start, width))
        return (logits_ref[sl] + g) * inv_t, sl

    def p1_body(k, m):
        z, sl = z_chunk(k * CHUNK, CHUNK)
        out_ref[sl] = z
        return jnp.maximum(m, jnp.max(z, axis=-1, keepdims=True))

    m0 = jnp.full((BLOCK_ROWS, 1), -jnp.inf, jnp.float32)
    m = jax.lax.fori_loop(0, NFULL, p1_body, m0)
    z, sl = z_chunk(TAIL_START, TAIL)
    out_ref[sl] = z
    m = jnp.maximum(m, jnp.max(z, axis=-1, keepdims=True))

    def p2_body(k, s):
        sl = (slice(None), pl.ds(k * CHUNK, CHUNK))
        e = jnp.exp(out_ref[sl] - m)
        out_ref[sl] = e
        return s + jnp.sum(e, axis=-1, keepdims=True)

    s = jax.lax.fori_loop(0, NFULL, p2_body, jnp.zeros((BLOCK_ROWS, 1), jnp.float32))
    sl = (slice(None), pl.ds(TAIL_START, TAIL))
    e = jnp.exp(out_ref[sl] - m)
    out_ref[sl] = e
    s = s + jnp.sum(e, axis=-1, keepdims=True)

    inv_s = 1.0 / s

    def p3_body(k, carry):
        sl = (slice(None), pl.ds(k * CHUNK, CHUNK))
        out_ref[sl] = out_ref[sl] * inv_s
        return carry

    jax.lax.fori_loop(0, NFULL, p3_body, 0)
    sl = (slice(None), pl.ds(TAIL_START, TAIL))
    out_ref[sl] = out_ref[sl] * inv_s


@jax.jit
def kernel(logits, temperature):
    inv_t = (1.0 / jnp.asarray(temperature, jnp.float32)).reshape(1)
    grid = (ROWS // BLOCK_ROWS,)
    return pl.pallas_call(
        _gumbel_softmax_kernel,
        grid=grid,
        in_specs=[
            pl.BlockSpec(memory_space=pltpu.SMEM),
            pl.BlockSpec((BLOCK_ROWS, COLS), lambda b: (b, 0)),
        ],
        out_specs=pl.BlockSpec((BLOCK_ROWS, COLS), lambda b: (b, 0)),
        out_shape=jax.ShapeDtypeStruct((ROWS, COLS), jnp.float32),
    )(inv_t, logits)


# speedup vs baseline: 1.3609x; 1.3304x over previous
import jax
import jax.numpy as jnp
from jax.experimental import pallas as pl
from jax.experimental.pallas import tpu as pltpu

ROWS, COLS = 128, 100000
BLOCK_ROWS = 8
CHUNK = 4096
NFULL = COLS // CHUNK
TAIL_START = NFULL * CHUNK
TAIL = COLS - TAIL_START


def _rotl(x, d):
    return (x << jnp.uint32(d)) | (x >> jnp.uint32(32 - d))


def _bits_from_idx(idx):
    ks0 = jnp.uint32(0)
    ks1 = jnp.uint32(1)
    ks2 = jnp.uint32(0x1BD11BDB)
    ks = (ks0, ks1, ks2)
    rotations = ((13, 15, 26, 6), (17, 29, 16, 24))
    x1 = idx + ks1
    x0 = x1
    x1 = _rotl(x1, 13)
    x1 = x0 ^ x1
    for r in (15, 26, 6):
        x0 = x0 + x1
        x1 = _rotl(x1, r)
        x1 = x0 ^ x1
    x0 = x0 + ks[1]
    x1 = x1 + ks[2] + jnp.uint32(1)
    for i in range(1, 5):
        for r in rotations[i % 2]:
            x0 = x0 + x1
            x1 = _rotl(x1, r)
            x1 = x0 ^ x1
        x0 = x0 + ks[(i + 1) % 3]
        x1 = x1 + ks[(i + 2) % 3] + jnp.uint32(i + 1)
    return x0 ^ x1


def _gumbel(idx):
    bits = _bits_from_idx(idx)
    fbits = (bits >> jnp.uint32(9)) | jnp.uint32(0x3F800000)
    u = jax.lax.bitcast_convert_type(fbits, jnp.float32) - 1.0
    eps = jnp.float32(1e-20)
    return -jnp.log(-jnp.log(u + eps) + eps)


def _gumbel_softmax_kernel(inv_t_ref, logits_ref, out_ref):
    b = pl.program_id(0)
    inv_t = inv_t_ref[0]
    base = jnp.uint32(b) * jnp.uint32(BLOCK_ROWS * COLS)

    def idx_chunk(start, width):
        shape = (BLOCK_ROWS, width)
        row = jax.lax.broadcasted_iota(jnp.uint32, shape, 0)
        col = jax.lax.broadcasted_iota(jnp.uint32, shape, 1)
        start_u = jnp.asarray(start, jnp.int32).astype(jnp.uint32)
        return base + row * jnp.uint32(COLS) + (col + start_u)

    def z_chunk(start, width):
        sl = (slice(None), pl.ds(start, width))
        g = _gumbel(idx_chunk(start, width))
        return (logits_ref[sl] + g) * inv_t, sl

    def p1_body(k, m):
        z, sl = z_chunk(k * CHUNK, CHUNK)
        out_ref[sl] = z
        return jnp.maximum(m, z)

    m0 = jnp.full((BLOCK_ROWS, CHUNK), -jnp.inf, jnp.float32)
    mc = jax.lax.fori_loop(0, NFULL, p1_body, m0)
    m = jnp.max(mc, axis=-1, keepdims=True)
    z, sl = z_chunk(TAIL_START, TAIL)
    out_ref[sl] = z
    m = jnp.maximum(m, jnp.max(z, axis=-1, keepdims=True))

    def p2_body(k, s):
        sl = (slice(None), pl.ds(k * CHUNK, CHUNK))
        e = jnp.exp(out_ref[sl] - m)
        out_ref[sl] = e
        return s + e

    sc = jax.lax.fori_loop(0, NFULL, p2_body, jnp.zeros((BLOCK_ROWS, CHUNK), jnp.float32))
    s = jnp.sum(sc, axis=-1, keepdims=True)
    sl = (slice(None), pl.ds(TAIL_START, TAIL))
    e = jnp.exp(out_ref[sl] - m)
    out_ref[sl] = e
    s = s + jnp.sum(e, axis=-1, keepdims=True)

    inv_s = 1.0 / s

    def p3_body(k, carry):
        sl = (slice(None), pl.ds(k * CHUNK, CHUNK))
        out_ref[sl] = out_ref[sl] * inv_s
        return carry

    jax.lax.fori_loop(0, NFULL, p3_body, 0)
    sl = (slice(None), pl.ds(TAIL_START, TAIL))
    out_ref[sl] = out_ref[sl] * inv_s


@jax.jit
def kernel(logits, temperature):
    inv_t = (1.0 / jnp.asarray(temperature, jnp.float32)).reshape(1)
    grid = (ROWS // BLOCK_ROWS,)
    return pl.pallas_call(
        _gumbel_softmax_kernel,
        grid=grid,
        in_specs=[
            pl.BlockSpec(memory_space=pltpu.SMEM),
            pl.BlockSpec((BLOCK_ROWS, COLS), lambda b: (b, 0)),
        ],
        out_specs=pl.BlockSpec((BLOCK_ROWS, COLS), lambda b: (b, 0)),
        out_shape=jax.ShapeDtypeStruct((ROWS, COLS), jnp.float32),
    )(inv_t, logits)


# speedup vs baseline: 1.7216x; 1.2650x over previous
import jax
import jax.numpy as jnp
from jax.experimental import pallas as pl
from jax.experimental.pallas import tpu as pltpu

ROWS, COLS = 128, 100000
BLK = 1000
SUB = 200
NBLK = COLS // BLK
NSUB = BLK // SUB


def _rotl(x, d):
    return (x << jnp.uint32(d)) | (x >> jnp.uint32(32 - d))


def _bits_from_idx(idx):
    ks = (jnp.uint32(0), jnp.uint32(1), jnp.uint32(0x1BD11BDB))
    rotations = ((13, 15, 26, 6), (17, 29, 16, 24))
    x1 = idx + ks[1]
    x0 = x1
    x1 = _rotl(x1, 13)
    x1 = x0 ^ x1
    for r in (15, 26, 6):
        x0 = x0 + x1
        x1 = _rotl(x1, r)
        x1 = x0 ^ x1
    x0 = x0 + ks[1]
    x1 = x1 + ks[2] + jnp.uint32(1)
    for i in range(1, 5):
        for r in rotations[i % 2]:
            x0 = x0 + x1
            x1 = _rotl(x1, r)
            x1 = x0 ^ x1
        x0 = x0 + ks[(i + 1) % 3]
        x1 = x1 + ks[(i + 2) % 3] + jnp.uint32(i + 1)
    return x0 ^ x1


def _gumbel(idx):
    bits = _bits_from_idx(idx)
    fbits = (bits >> jnp.uint32(9)) | jnp.uint32(0x3F800000)
    u = jax.lax.bitcast_convert_type(fbits, jnp.float32) - 1.0
    eps = jnp.float32(1e-20)
    return -jnp.log(-jnp.log(u + eps) + eps)


def _kernel_body(inv_t_ref, lt_ref, out_ref, z_buf, acc_m, acc_s):
    p = pl.program_id(0)
    k = pl.program_id(1)
    row0 = k * BLK

    @pl.when(p == 0)
    def _phase0():
        inv_t = inv_t_ref[0]
        base = jnp.asarray(row0, jnp.int32).astype(jnp.uint32)
        m8 = jnp.full((8, 128), -jnp.inf, jnp.float32)
        for j in range(NSUB):
            shape = (SUB, 128)
            r_io = jax.lax.broadcasted_iota(jnp.uint32, shape, 0)
            c_io = jax.lax.broadcasted_iota(jnp.uint32, shape, 1)
            idx = (base + jnp.uint32(j * SUB) + r_io) + c_io * jnp.uint32(COLS)
            g = _gumbel(idx)
            z = (lt_ref[pl.ds(j * SUB, SUB), :] + g) * inv_t
            z_buf[pl.ds(row0 + j * SUB, SUB), :] = z
            m8 = jnp.maximum(m8, jnp.max(z.reshape(SUB // 8, 8, 128), axis=0))
        @pl.when(k == 0)
        def _():
            acc_m[...] = m8
        @pl.when(k != 0)
        def _():
            acc_m[...] = jnp.maximum(acc_m[...], m8)

    @pl.when(p == 1)
    def _phase1():
        m = jnp.max(acc_m[...], axis=0, keepdims=True)
        s8 = jnp.zeros((8, 128), jnp.float32)
        for j in range(NSUB):
            sl = (pl.ds(row0 + j * SUB, SUB), slice(None))
            e = jnp.exp(z_buf[sl] - m)
            z_buf[sl] = e
            s8 = s8 + jnp.sum(e.reshape(SUB // 8, 8, 128), axis=0)
        @pl.when(k == 0)
        def _():
            acc_s[...] = s8
        @pl.when(k != 0)
        def _():
            acc_s[...] = acc_s[...] + s8

    @pl.when(p == 2)
    def _phase2():
        inv_s = 1.0 / jnp.sum(acc_s[...], axis=0, keepdims=True)
        for j in range(NSUB):
            out_ref[pl.ds(j * SUB, SUB), :] = (
                z_buf[pl.ds(row0 + j * SUB, SUB), :] * inv_s)


@jax.jit
def kernel(logits, temperature):
    inv_t = (1.0 / jnp.asarray(temperature, jnp.float32)).reshape(1)
    lt = logits.T
    out_t = pl.pallas_call(
        _kernel_body,
        grid=(3, NBLK),
        in_specs=[
            pl.BlockSpec(memory_space=pltpu.SMEM),
            pl.BlockSpec((BLK, ROWS), lambda p, k: (jnp.where(p == 0, k, 0), 0)),
        ],
        out_specs=pl.BlockSpec((BLK, ROWS), lambda p, k: (jnp.where(p == 2, k, 0), 0)),
        out_shape=jax.ShapeDtypeStruct((COLS, ROWS), jnp.float32),
        scratch_shapes=[
            pltpu.VMEM((COLS, ROWS), jnp.float32),
            pltpu.VMEM((8, 128), jnp.float32),
            pltpu.VMEM((8, 128), jnp.float32),
        ],
    )(inv_t, lt)
    return out_t.T


# speedup vs baseline: 1.8887x; 1.0970x over previous
import jax
import jax.numpy as jnp
from jax.experimental import pallas as pl
from jax.experimental.pallas import tpu as pltpu

ROWS, COLS = 128, 100000
BLK = 2000
SUB = 200
NBLK = COLS // BLK
NSUB = BLK // SUB


def _rotl(x, d):
    return (x << jnp.uint32(d)) | (x >> jnp.uint32(32 - d))


def _bits_from_idx(idx):
    ks = (jnp.uint32(0), jnp.uint32(1), jnp.uint32(0x1BD11BDB))
    rotations = ((13, 15, 26, 6), (17, 29, 16, 24))
    x1 = idx + ks[1]
    x0 = x1
    x1 = _rotl(x1, 13)
    x1 = x0 ^ x1
    for r in (15, 26, 6):
        x0 = x0 + x1
        x1 = _rotl(x1, r)
        x1 = x0 ^ x1
    x0 = x0 + ks[1]
    x1 = x1 + ks[2] + jnp.uint32(1)
    for i in range(1, 5):
        for r in rotations[i % 2]:
            x0 = x0 + x1
            x1 = _rotl(x1, r)
            x1 = x0 ^ x1
        x0 = x0 + ks[(i + 1) % 3]
        x1 = x1 + ks[(i + 2) % 3] + jnp.uint32(i + 1)
    return x0 ^ x1


def _gumbel(idx):
    bits = _bits_from_idx(idx)
    fbits = (bits >> jnp.uint32(9)) | jnp.uint32(0x3F800000)
    u = jax.lax.bitcast_convert_type(fbits, jnp.float32) - 1.0
    eps = jnp.float32(1e-20)
    return -jnp.log(-jnp.log(u + eps) + eps)


def _kernel_body(inv_t_ref, lt_ref, out_ref, z_buf, acc_m, acc_s):
    p = pl.program_id(0)
    k = pl.program_id(1)
    row0 = k * BLK

    @pl.when(p == 0)
    def _phase0():
        inv_t = inv_t_ref[0]
        base = jnp.asarray(row0, jnp.int32).astype(jnp.uint32)
        m8 = jnp.full((8, 128), -jnp.inf, jnp.float32)
        for j in range(NSUB):
            shape = (SUB, 128)
            r_io = jax.lax.broadcasted_iota(jnp.uint32, shape, 0)
            c_io = jax.lax.broadcasted_iota(jnp.uint32, shape, 1)
            idx = (base + jnp.uint32(j * SUB) + r_io) + c_io * jnp.uint32(COLS)
            g = _gumbel(idx)
            z = (lt_ref[pl.ds(j * SUB, SUB), :] + g) * inv_t
            z_buf[pl.ds(row0 + j * SUB, SUB), :] = z
            m8 = jnp.maximum(m8, jnp.max(z.reshape(SUB // 8, 8, 128), axis=0))
        @pl.when(k == 0)
        def _():
            acc_m[...] = m8
        @pl.when(k != 0)
        def _():
            acc_m[...] = jnp.maximum(acc_m[...], m8)

    @pl.when(p == 1)
    def _phase1():
        m = jnp.max(acc_m[...], axis=0, keepdims=True)
        s8 = jnp.zeros((8, 128), jnp.float32)
        for j in range(NSUB):
            sl = (pl.ds(row0 + j * SUB, SUB), slice(None))
            e = jnp.exp(z_buf[sl] - m)
            z_buf[sl] = e
            s8 = s8 + jnp.sum(e.reshape(SUB // 8, 8, 128), axis=0)
        @pl.when(k == 0)
        def _():
            acc_s[...] = s8
        @pl.when(k != 0)
        def _():
            acc_s[...] = acc_s[...] + s8

    @pl.when(p == 2)
    def _phase2():
        inv_s = 1.0 / jnp.sum(acc_s[...], axis=0, keepdims=True)
        for j in range(NSUB):
            out_ref[pl.ds(j * SUB, SUB), :] = (
                z_buf[pl.ds(row0 + j * SUB, SUB), :] * inv_s)


@jax.jit
def kernel(logits, temperature):
    inv_t = (1.0 / jnp.asarray(temperature, jnp.float32)).reshape(1)
    lt = logits.T
    out_t = pl.pallas_call(
        _kernel_body,
        grid=(3, NBLK),
        in_specs=[
            pl.BlockSpec(memory_space=pltpu.SMEM),
            pl.BlockSpec((BLK, ROWS), lambda p, k: (jnp.where(p == 0, k, 0), 0)),
        ],
        out_specs=pl.BlockSpec((BLK, ROWS), lambda p, k: (jnp.where(p == 2, k, 0), 0)),
        out_shape=jax.ShapeDtypeStruct((COLS, ROWS), jnp.float32),
        scratch_shapes=[
            pltpu.VMEM((COLS, ROWS), jnp.float32),
            pltpu.VMEM((8, 128), jnp.float32),
            pltpu.VMEM((8, 128), jnp.float32),
        ],
    )(inv_t, lt)
    return out_t.T
